# Initial kernel scaffold; baseline (speedup 1.0000x reference)
import jax
import jax.numpy as jnp
from jax.experimental import pallas as pl


def kernel(x1, x2, x3, Wq_12, bq_12, Wk_12, bk_12, Wq_13, bq_13, Wk_13, bk_13, Wq_21, bq_21, Wk_21, bk_21, Wq_23, bq_23, Wk_23, bk_23, Wq_31, bq_31, Wk_31, bk_31, Wq_32, bq_32, Wk_32, bk_32, mw1, mw2, mw3):
    raise NotImplementedError("write your pallas kernel here")


# speedup vs baseline: 1.9249x; 1.9249x over previous
import functools

import jax
import jax.numpy as jnp
from jax import lax
from jax.experimental import pallas as pl
from jax.experimental.pallas import tpu as pltpu
from jax.experimental.pallas import tpu_sc as plsc

B, N, C, H = 8, 4096, 256, 4
HD = C // H
SCALE = HD ** -0.5
KEEP = 2458
ROWS = 3 * B
_BITS_HI = 0x3F800010


_PAIR_IDX = {(0, 1): 0, (0, 2): 1, (1, 0): 2, (1, 2): 3, (2, 0): 4, (2, 1): 5}


def _scores_body(x1_ref, x2_ref, x3_ref, wq_ref, bqt_ref, wk_ref, bkt_ref,
                 ws_ref, s_ref):
    xs = (x1_ref[0], x2_ref[0], x3_ref[0])
    gcols, gncols = [], []
    for x in xs:
        g = jnp.mean(x, axis=0, keepdims=True)
        gc = g.T
        nrm = jnp.sqrt(jnp.sum(gc * gc)) + 1e-8
        gcols.append(gc)
        gncols.append(gc / nrm)

    rowh = lax.broadcasted_iota(jnp.int32, (C, H), 0) // HD
    colh = lax.broadcasted_iota(jnp.int32, (C, H), 1)
    hmask = (rowh == colh).astype(jnp.float32)

    ws = ws_ref[...]

    for m in range(3):
        o1, o2 = [o for o in range(3) if o != m]
        x = xs[m]
        r = 1.0 / (jnp.sqrt(jnp.sum(x * x, axis=1, keepdims=True)) + 1e-8)
        cols = [gncols[m], gncols[o1], gncols[o2]]
        consts = []
        for o in (o1, o2):
            p = _PAIR_IDX[(m, o)]
            wq = wq_ref[p]
            bqc = bqt_ref[:, p:p + 1]
            qc = lax.dot_general(wq, gcols[o], (((1,), (0,)), ((), ())),
                                 precision=lax.Precision.HIGHEST) + bqc
            wk = wk_ref[p]
            a = wk * qc
            v = lax.dot_general(a, hmask, (((0,), (0,)), ((), ())),
                                precision=lax.Precision.HIGHEST) * SCALE
            bkq = bkt_ref[:, p:p + 1] * qc
            c0 = lax.dot_general(bkq, hmask, (((0,), (0,)), ((), ())),
                                 precision=lax.Precision.HIGHEST) * SCALE
            cols.append(v)
            consts.append(c0)
        mcols = jnp.concatenate(cols, axis=1)
        p_all = lax.dot_general(x, mcols, (((1,), (0,)), ((), ())),
                                precision=lax.Precision.HIGHEST)

        parts = [p_all[:, 0:1] * r]
        for i in range(2):
            cos_o = p_all[:, 1 + i:2 + i] * r
            lg = p_all[:, 3 + i * H:3 + (i + 1) * H] + consts[i]
            mx = jnp.max(lg, axis=0, keepdims=True)
            e = jnp.exp(lg - mx)
            z = jnp.sum(e, axis=0, keepdims=True)
            amean = jnp.mean(e / z, axis=1, keepdims=True)
            parts.append(cos_o * amean)

        fused = jnp.zeros((N, 1), jnp.float32)
        for j in range(3):
            sj = parts[j]
            mn = jnp.min(sj, axis=0, keepdims=True)
            mx = jnp.max(sj, axis=0, keepdims=True)
            fused = fused + ws[m:m + 1, j:j + 1] * ((sj - mn) / (mx - mn + 1e-8))
        s_ref[m, 0] = fused


_scores_call = pl.pallas_call(
    _scores_body,
    grid=(B,),
    in_specs=[
        pl.BlockSpec((1, N, C), lambda b: (b, 0, 0)),
        pl.BlockSpec((1, N, C), lambda b: (b, 0, 0)),
        pl.BlockSpec((1, N, C), lambda b: (b, 0, 0)),
        pl.BlockSpec((6, C, C), lambda b: (0, 0, 0)),
        pl.BlockSpec((C, 6), lambda b: (0, 0)),
        pl.BlockSpec((6, C, C), lambda b: (0, 0, 0)),
        pl.BlockSpec((C, 6), lambda b: (0, 0)),
        pl.BlockSpec((3, 3), lambda b: (0, 0)),
    ],
    out_specs=pl.BlockSpec((3, 1, N, 1), lambda b: (0, b, 0, 0)),
    out_shape=jax.ShapeDtypeStruct((3, B, N, 1), jnp.float32),
    compiler_params=pltpu.CompilerParams(vmem_limit_bytes=100 * 1024 * 1024),
)


def _topk_body(scores_hbm, out_hbm, score_vm, thr_vm):
    row = lax.axis_index("s") * 2 + lax.axis_index("c")

    @pl.when(row < ROWS)
    def _():
        pltpu.sync_copy(scores_hbm.at[row], score_vm)

        def outer(_, lohi):
            lo, hi = lohi
            mid = lax.shift_right_logical(lo + hi, 1)

            def inner(j, cnt):
                bits = plsc.bitcast(score_vm[pl.ds(j * 16, 16)], jnp.int32)
                return cnt + plsc.all_reduce_population_count(bits > mid)

            cnt = lax.fori_loop(0, N // 16, inner, jnp.zeros((16,), jnp.int32))
            pred = cnt >= KEEP
            return jnp.where(pred, mid, lo), jnp.where(pred, hi, mid)

        lo0 = jnp.zeros((16,), jnp.int32)
        hi0 = jnp.full((16,), _BITS_HI, jnp.int32)
        lo, _ = lax.fori_loop(0, 32, outer, (lo0, hi0))
        thr_vm[...] = plsc.bitcast(lo, jnp.float32)
        pltpu.sync_copy(thr_vm, out_hbm.at[row])


@functools.cache
def _topk_call():
    return pl.kernel(
        _topk_body,
        out_type=jax.ShapeDtypeStruct((ROWS, 16), jnp.float32),
        mesh=plsc.VectorSubcoreMesh(core_axis_name="c", subcore_axis_name="s"),
        scratch_types=[
            pltpu.VMEM((N,), jnp.float32),
            pltpu.VMEM((16,), jnp.float32),
        ],
        compiler_params=pltpu.CompilerParams(needs_layout_passes=False),
    )


_TN = 1024


def _mask_body(x1_ref, x2_ref, x3_ref, s_ref, thr_ref, o_ref):
    b = pl.program_id(0)
    for m, xref in enumerate((x1_ref, x2_ref, x3_ref)):
        t = thr_ref[pl.ds(b, 1), m:m + 1]
        maskf = (s_ref[m, 0] > t).astype(jnp.float32)
        o_ref[m, 0] = xref[0] * maskf


_mask_call = pl.pallas_call(
    _mask_body,
    grid=(B, N // _TN),
    in_specs=[
        pl.BlockSpec((1, _TN, C), lambda b, n: (b, n, 0)),
        pl.BlockSpec((1, _TN, C), lambda b, n: (b, n, 0)),
        pl.BlockSpec((1, _TN, C), lambda b, n: (b, n, 0)),
        pl.BlockSpec((3, 1, _TN, 1), lambda b, n: (0, b, n, 0)),
        pl.BlockSpec((B, 3), lambda b, n: (0, 0)),
    ],
    out_specs=pl.BlockSpec((3, 1, _TN, C), lambda b, n: (0, b, n, 0)),
    out_shape=jax.ShapeDtypeStruct((3, B, N, C), jnp.float32),
)


def kernel(x1, x2, x3, Wq_12, bq_12, Wk_12, bk_12, Wq_13, bq_13, Wk_13, bk_13,
           Wq_21, bq_21, Wk_21, bk_21, Wq_23, bq_23, Wk_23, bk_23,
           Wq_31, bq_31, Wk_31, bk_31, Wq_32, bq_32, Wk_32, bk_32,
           mw1, mw2, mw3):
    wqs = jnp.stack([Wq_12, Wq_13, Wq_21, Wq_23, Wq_31, Wq_32])
    wks = jnp.stack([Wk_12, Wk_13, Wk_21, Wk_23, Wk_31, Wk_32])
    bqt = jnp.stack([bq_12, bq_13, bq_21, bq_23, bq_31, bq_32], axis=1)
    bkt = jnp.stack([bk_12, bk_13, bk_21, bk_23, bk_31, bk_32], axis=1)
    ws = jax.nn.softmax(jnp.stack([mw1, mw2, mw3]), axis=-1)

    scores = _scores_call(x1, x2, x3, wqs, bqt, wks, bkt, ws)
    thr24 = _topk_call()(scores.reshape(ROWS, N))
    thr = thr24[:, 0].reshape(3, B).T
    return _mask_call(x1, x2, x3, scores, thr)


# speedup vs baseline: 2.0356x; 1.0575x over previous
import functools

import jax
import jax.numpy as jnp
from jax import lax
from jax.experimental import pallas as pl
from jax.experimental.pallas import tpu as pltpu
from jax.experimental.pallas import tpu_sc as plsc

B, N, C, H = 8, 4096, 256, 4
HD = C // H
SCALE = HD ** -0.5
KEEP = 2458
ROWS = 3 * B
_BITS_HI = 0x3F800010


_PAIR_IDX = {(0, 1): 0, (0, 2): 1, (1, 0): 2, (1, 2): 3, (2, 0): 4, (2, 1): 5}


def _scores_body(x1_ref, x2_ref, x3_ref, wq_ref, bqt_ref, wk_ref, bkt_ref,
                 ws_ref, s_ref):
    xs = (x1_ref[0], x2_ref[0], x3_ref[0])
    gcols, gncols = [], []
    for x in xs:
        g = jnp.mean(x, axis=0, keepdims=True)
        gc = g.T
        nrm = jnp.sqrt(jnp.sum(gc * gc)) + 1e-8
        gcols.append(gc)
        gncols.append(gc / nrm)

    rowh = lax.broadcasted_iota(jnp.int32, (C, H), 0) // HD
    colh = lax.broadcasted_iota(jnp.int32, (C, H), 1)
    hmask = (rowh == colh).astype(jnp.float32)

    ws = ws_ref[...]

    for m in range(3):
        o1, o2 = [o for o in range(3) if o != m]
        x = xs[m]
        r = 1.0 / (jnp.sqrt(jnp.sum(x * x, axis=1, keepdims=True)) + 1e-8)
        cols = [gncols[m], gncols[o1], gncols[o2]]
        consts = []
        for o in (o1, o2):
            p = _PAIR_IDX[(m, o)]
            wq = wq_ref[p]
            bqc = bqt_ref[:, p:p + 1]
            qc = lax.dot_general(wq, gcols[o], (((1,), (0,)), ((), ())),
                                 precision=lax.Precision.HIGHEST) + bqc
            wk = wk_ref[p]
            a = wk * qc
            v = lax.dot_general(a, hmask, (((0,), (0,)), ((), ())),
                                precision=lax.Precision.HIGHEST) * SCALE
            bkq = bkt_ref[:, p:p + 1] * qc
            c0 = lax.dot_general(bkq, hmask, (((0,), (0,)), ((), ())),
                                 precision=lax.Precision.HIGHEST) * SCALE
            cols.append(v)
            consts.append(c0)
        mcols = jnp.concatenate(cols, axis=1)
        p_all = lax.dot_general(x, mcols, (((1,), (0,)), ((), ())),
                                precision=lax.Precision.HIGHEST)

        parts = [p_all[:, 0:1] * r]
        for i in range(2):
            cos_o = p_all[:, 1 + i:2 + i] * r
            lg = p_all[:, 3 + i * H:3 + (i + 1) * H] + consts[i]
            mx = jnp.max(lg, axis=0, keepdims=True)
            e = jnp.exp(lg - mx)
            z = jnp.sum(e, axis=0, keepdims=True)
            amean = jnp.mean(e / z, axis=1, keepdims=True)
            parts.append(cos_o * amean)

        fused = jnp.zeros((N, 1), jnp.float32)
        for j in range(3):
            sj = parts[j]
            mn = jnp.min(sj, axis=0, keepdims=True)
            mx = jnp.max(sj, axis=0, keepdims=True)
            fused = fused + ws[m:m + 1, j:j + 1] * ((sj - mn) / (mx - mn + 1e-8))
        s_ref[m, 0] = fused.T


_scores_call = pl.pallas_call(
    _scores_body,
    grid=(B,),
    in_specs=[
        pl.BlockSpec((1, N, C), lambda b: (b, 0, 0)),
        pl.BlockSpec((1, N, C), lambda b: (b, 0, 0)),
        pl.BlockSpec((1, N, C), lambda b: (b, 0, 0)),
        pl.BlockSpec((6, C, C), lambda b: (0, 0, 0)),
        pl.BlockSpec((C, 6), lambda b: (0, 0)),
        pl.BlockSpec((6, C, C), lambda b: (0, 0, 0)),
        pl.BlockSpec((C, 6), lambda b: (0, 0)),
        pl.BlockSpec((3, 3), lambda b: (0, 0)),
    ],
    out_specs=pl.BlockSpec((3, 1, 1, N), lambda b: (0, b, 0, 0)),
    out_shape=jax.ShapeDtypeStruct((3, B, 1, N), jnp.float32),
    compiler_params=pltpu.CompilerParams(vmem_limit_bytes=100 * 1024 * 1024),
)


def _topk_body(scores_hbm, out_hbm, score_vm, thr_vm):
    row = lax.axis_index("s") * 2 + lax.axis_index("c")

    @pl.when(row < ROWS)
    def _():
        pltpu.sync_copy(scores_hbm.at[row], score_vm)

        one = jnp.full((16,), 1, jnp.int32)
        zero = jnp.zeros((16,), jnp.int32)

        def outer(_, lohi):
            lo, hi = lohi
            mid = lax.shift_right_logical(lo + hi, 1)

            def inner(j, cnt):
                bits = plsc.bitcast(score_vm[pl.ds(j * 16, 16)], jnp.int32)
                return cnt + jnp.where(bits > mid, one, zero)

            cntv = lax.fori_loop(0, N // 16, inner, zero)
            tot = jnp.sum(cntv, axis=0)
            pred = jnp.full((16,), 1, jnp.int32) * tot >= KEEP
            return jnp.where(pred, mid, lo), jnp.where(pred, hi, mid)

        lo0 = jnp.zeros((16,), jnp.int32)
        hi0 = jnp.full((16,), _BITS_HI, jnp.int32)
        lo, _ = lax.fori_loop(0, 32, outer, (lo0, hi0))
        thr_vm[...] = plsc.bitcast(lo, jnp.float32)
        pltpu.sync_copy(thr_vm, out_hbm.at[row])


@functools.cache
def _topk_call():
    return pl.kernel(
        _topk_body,
        out_type=jax.ShapeDtypeStruct((ROWS, 16), jnp.float32),
        mesh=plsc.VectorSubcoreMesh(core_axis_name="c", subcore_axis_name="s"),
        scratch_types=[
            pltpu.VMEM((N,), jnp.float32),
            pltpu.VMEM((16,), jnp.float32),
        ],
        compiler_params=pltpu.CompilerParams(needs_layout_passes=False),
    )


_TN = 1024


def _mask_body(x1_ref, x2_ref, x3_ref, s_ref, thr_ref, o_ref):
    b = pl.program_id(0)
    for m, xref in enumerate((x1_ref, x2_ref, x3_ref)):
        t = thr_ref[pl.ds(b, 1), m:m + 1]
        sv = s_ref[m, 0].T
        maskf = (sv > t).astype(jnp.float32)
        o_ref[m, 0] = xref[0] * maskf


_mask_call = pl.pallas_call(
    _mask_body,
    grid=(B, N // _TN),
    in_specs=[
        pl.BlockSpec((1, _TN, C), lambda b, n: (b, n, 0)),
        pl.BlockSpec((1, _TN, C), lambda b, n: (b, n, 0)),
        pl.BlockSpec((1, _TN, C), lambda b, n: (b, n, 0)),
        pl.BlockSpec((3, 1, 1, _TN), lambda b, n: (0, b, 0, n)),
        pl.BlockSpec((B, 3), lambda b, n: (0, 0)),
    ],
    out_specs=pl.BlockSpec((3, 1, _TN, C), lambda b, n: (0, b, n, 0)),
    out_shape=jax.ShapeDtypeStruct((3, B, N, C), jnp.float32),
)


def kernel(x1, x2, x3, Wq_12, bq_12, Wk_12, bk_12, Wq_13, bq_13, Wk_13, bk_13,
           Wq_21, bq_21, Wk_21, bk_21, Wq_23, bq_23, Wk_23, bk_23,
           Wq_31, bq_31, Wk_31, bk_31, Wq_32, bq_32, Wk_32, bk_32,
           mw1, mw2, mw3):
    wqs = jnp.stack([Wq_12, Wq_13, Wq_21, Wq_23, Wq_31, Wq_32])
    wks = jnp.stack([Wk_12, Wk_13, Wk_21, Wk_23, Wk_31, Wk_32])
    bqt = jnp.stack([bq_12, bq_13, bq_21, bq_23, bq_31, bq_32], axis=1)
    bkt = jnp.stack([bk_12, bk_13, bk_21, bk_23, bk_31, bk_32], axis=1)
    ws = jax.nn.softmax(jnp.stack([mw1, mw2, mw3]), axis=-1)

    scores = _scores_call(x1, x2, x3, wqs, bqt, wks, bkt, ws)
    thr24 = _topk_call()(scores.reshape(ROWS, N))
    thr = thr24[:, 0].reshape(3, B).T
    return _mask_call(x1, x2, x3, scores, thr)
